# baseline (device time: 8258 ns/iter reference)
import jax
import jax.numpy as jnp
from jax import lax
from jax.experimental import pallas as pl
from jax.experimental.pallas import tpu as pltpu

CHUNK = 128


def kernel(x, dest):
    m, n = x.shape
    f32 = jnp.float32
    bf16 = jnp.bfloat16
    n_chunks = m // CHUNK

    def body(x_hbm, dest_ref, out_hbm, xv_ref, outv_ref, send_ref, comm_ref,
             in_sem, out_sem, send_sems, recv_sems):
        my_x = lax.axis_index("x")
        my_y = lax.axis_index("y")
        my_z = lax.axis_index("z")
        partner = (1 - my_x, my_y, my_z)
        is0 = my_x == 0

        copy_in = pltpu.make_async_copy(x_hbm, xv_ref, in_sem)
        copy_in.start()

        barrier_sem = pltpu.get_barrier_semaphore()
        pl.semaphore_signal(
            barrier_sem, inc=1, device_id=partner,
            device_id_type=pl.DeviceIdType.MESH,
        )

        row_i = lax.broadcasted_iota(jnp.int32, (m, m), 0)
        col_i = lax.broadcasted_iota(jnp.int32, (m, m), 1)
        U = (row_i < col_i).astype(f32)
        ind0 = (dest_ref[:, :] == 0).astype(f32)
        c0 = jnp.dot(ind0, U, preferred_element_type=f32)
        i_vec = lax.broadcasted_iota(jnp.int32, (1, m), 1).astype(f32)
        n0 = jnp.sum(ind0)
        k = jnp.where(is0, m - n0, n0)

        ind_send = (dest_ref[:, :] == 1 - my_x).astype(f32)
        send_pos = jnp.where(is0, i_vec - c0, (m - k) + c0)
        send_slot = jnp.where(ind_send > 0, send_pos, -1.0).astype(jnp.int32)

        pl.semaphore_wait(barrier_sem, 1)
        copy_in.wait()

        rdmas = []
        for c in range(n_chunks):
            rdma = pltpu.make_async_remote_copy(
                src_ref=send_ref.at[pl.ds(c * CHUNK, CHUNK)],
                dst_ref=comm_ref.at[pl.ds(c * CHUNK, CHUNK)],
                send_sem=send_sems.at[c],
                recv_sem=recv_sems.at[c],
                device_id=partner, device_id_type=pl.DeviceIdType.MESH,
            )
            front_live = k > c * CHUNK
            end_live = k + (c + 1) * CHUNK > m
            send_live = jnp.where(is0, front_live, end_live)
            recv_live = jnp.where(is0, end_live, front_live)

            @pl.when(send_live)
            def _(c=c, rdma=rdma):
                S_c = (row_i[c * CHUNK:(c + 1) * CHUNK, :]
                       == send_slot).astype(bf16)
                send_ref[pl.ds(c * CHUNK, CHUNK), :] = jnp.dot(
                    S_c, xv_ref[:, :], preferred_element_type=f32
                ).astype(bf16)
                rdma.start()

            rdmas.append((rdma, send_live, recv_live))

        ind_keep = (dest_ref[:, :] == my_x).astype(f32)
        keep_pos = jnp.where(is0, c0, n0 + i_vec - c0)
        keep_slot = jnp.where(ind_keep > 0, keep_pos, -1.0).astype(jnp.int32)
        K = (row_i == keep_slot).astype(bf16)
        keep = jnp.dot(K, xv_ref[:, :], preferred_element_type=f32)

        for rdma, send_live, recv_live in rdmas:
            pl.when(recv_live)(rdma.wait_recv)

        out_rows = lax.broadcasted_iota(jnp.int32, (m, n), 0).astype(f32)
        take_keep = (out_rows < n0) == is0
        outv_ref[:, :] = jnp.where(
            take_keep, keep, comm_ref[:, :].astype(f32)
        )
        copy_out = pltpu.make_async_copy(outv_ref, out_hbm, out_sem)
        copy_out.start()

        for rdma, send_live, recv_live in rdmas:
            pl.when(send_live)(rdma.wait_send)
        copy_out.wait()

    return pl.pallas_call(
        body,
        out_shape=jax.ShapeDtypeStruct((m, n), x.dtype),
        in_specs=[
            pl.BlockSpec(memory_space=pl.ANY),
            pl.BlockSpec(memory_space=pltpu.VMEM),
        ],
        out_specs=pl.BlockSpec(memory_space=pl.ANY),
        scratch_shapes=[
            pltpu.VMEM((m, n), jnp.float32),
            pltpu.VMEM((m, n), jnp.float32),
            pltpu.VMEM((m, n), bf16),
            pltpu.VMEM((m, n), bf16),
            pltpu.SemaphoreType.DMA,
            pltpu.SemaphoreType.DMA,
            pltpu.SemaphoreType.DMA((n_chunks,)),
            pltpu.SemaphoreType.DMA((n_chunks,)),
        ],
        compiler_params=pltpu.CompilerParams(collective_id=0),
    )(x, dest.reshape(1, m))


# device time: 8148 ns/iter; 1.0135x vs baseline; 1.0135x over previous
import jax
import jax.numpy as jnp
from jax import lax
from jax.experimental import pallas as pl
from jax.experimental.pallas import tpu as pltpu

CHUNK = 256


def kernel(x, dest):
    m, n = x.shape
    f32 = jnp.float32
    bf16 = jnp.bfloat16
    n_chunks = m // CHUNK

    def body(x_ref, dest_ref, out_ref, send_ref, comm_ref,
             send_sems, recv_sems):
        my_x = lax.axis_index("x")
        my_y = lax.axis_index("y")
        my_z = lax.axis_index("z")
        partner = (1 - my_x, my_y, my_z)
        is0 = my_x == 0

        barrier_sem = pltpu.get_barrier_semaphore()
        pl.semaphore_signal(
            barrier_sem, inc=1, device_id=partner,
            device_id_type=pl.DeviceIdType.MESH,
        )

        row_i = lax.broadcasted_iota(jnp.int32, (m, m), 0)
        col_i = lax.broadcasted_iota(jnp.int32, (m, m), 1)
        U = (row_i < col_i).astype(f32)
        ind0 = (dest_ref[:, :] == 0).astype(f32)
        c0 = jnp.dot(ind0, U, preferred_element_type=f32)
        i_vec = lax.broadcasted_iota(jnp.int32, (1, m), 1).astype(f32)
        n0 = jnp.sum(ind0)
        k = jnp.where(is0, m - n0, n0)

        ind_send = (dest_ref[:, :] == 1 - my_x).astype(f32)
        send_pos = jnp.where(is0, i_vec - c0, (m - k) + c0)
        send_slot = jnp.where(ind_send > 0, send_pos, -1.0).astype(jnp.int32)

        pl.semaphore_wait(barrier_sem, 1)

        rdmas = []
        for c in range(n_chunks):
            rdma = pltpu.make_async_remote_copy(
                src_ref=send_ref.at[pl.ds(c * CHUNK, CHUNK)],
                dst_ref=comm_ref.at[pl.ds(c * CHUNK, CHUNK)],
                send_sem=send_sems.at[c],
                recv_sem=recv_sems.at[c],
                device_id=partner, device_id_type=pl.DeviceIdType.MESH,
            )
            front_live = k > c * CHUNK
            end_live = k + (c + 1) * CHUNK > m
            send_live = jnp.where(is0, front_live, end_live)
            recv_live = jnp.where(is0, end_live, front_live)
            @pl.when(send_live)
            def _(c=c, rdma=rdma):
                S_c = (row_i[c * CHUNK:(c + 1) * CHUNK, :]
                       == send_slot).astype(bf16)
                send_ref[pl.ds(c * CHUNK, CHUNK), :] = jnp.dot(
                    S_c, x_ref[:, :], preferred_element_type=f32
                ).astype(bf16)
                rdma.start()

            rdmas.append((rdma, send_live, recv_live))

        ind_keep = (dest_ref[:, :] == my_x).astype(f32)
        keep_pos = jnp.where(is0, c0, n0 + i_vec - c0)
        keep_slot = jnp.where(ind_keep > 0, keep_pos, -1.0).astype(jnp.int32)
        K = (row_i == keep_slot).astype(bf16)
        keep = jnp.dot(K, x_ref[:, :], preferred_element_type=f32)

        for rdma, send_live, recv_live in rdmas:
            pl.when(recv_live)(rdma.wait_recv)

        out_rows = lax.broadcasted_iota(jnp.int32, (m, n), 0).astype(f32)
        take_keep = (out_rows < n0) == is0
        out_ref[:, :] = jnp.where(
            take_keep, keep, comm_ref[:, :].astype(f32)
        )

        for rdma, send_live, recv_live in rdmas:
            pl.when(send_live)(rdma.wait_send)

    return pl.pallas_call(
        body,
        out_shape=jax.ShapeDtypeStruct((m, n), x.dtype),
        in_specs=[
            pl.BlockSpec(memory_space=pltpu.VMEM),
            pl.BlockSpec(memory_space=pltpu.VMEM),
        ],
        out_specs=pl.BlockSpec(memory_space=pltpu.VMEM),
        scratch_shapes=[
            pltpu.VMEM((m, n), bf16),
            pltpu.VMEM((m, n), bf16),
            pltpu.SemaphoreType.DMA((n_chunks,)),
            pltpu.SemaphoreType.DMA((n_chunks,)),
        ],
        compiler_params=pltpu.CompilerParams(collective_id=0),
    )(x, dest.reshape(1, m))
